# baseline (speedup 1.0000x reference)
import functools

import jax
import jax.numpy as jnp
from jax import lax
from jax.experimental import pallas as pl
from jax.experimental.pallas import tpu as pltpu
from jax.experimental.pallas import tpu_sc as plsc

EMB = 64
NB = 2
NBIL = 8
NSPH = 7
NRAD = 6
CUT = 5.0
NBEF = 1
NAFT = 2
NDENSE = 3
NT = 12

NC = 2
NS = 16
NW = NC * NS

F32 = jnp.float32


def _swish(x):
    return x * jax.nn.sigmoid(x)


class _GatherStream:
    def __init__(self, tab, idx, out, I, R, semG, semW, base0, row0, hsl, C):
        self.tab, self.idx, self.out = tab, idx, out
        self.I, self.R, self.semG, self.semW = I, R, semG, semW
        self.base0, self.row0, self.hsl, self.C = base0, row0, hsl, C
        self.h_g = [None, None]
        self.h_w = [None, None]
        self.j = 0

    def _dst(self, j):
        r = self.row0 + j * self.C
        if self.hsl is None:
            return self.out.at[pl.ds(r, self.C)]
        return self.out.at[pl.ds(r, self.C), self.hsl]

    def step(self):
        j = self.j
        q = j & 1
        p = 1 - q
        if self.h_w[q] is not None:
            self.h_w[q].wait()
        pltpu.sync_copy(
            self.idx.at[pl.ds(self.base0 + j * self.C, self.C)], self.I[q])
        self.h_g[q] = pltpu.async_copy(
            self.tab.at[self.I[q]], self.R[q], self.semG[q])
        if j >= 1:
            self.h_g[p].wait()
            self.h_w[p] = pltpu.async_copy(
                self.R[p], self._dst(j - 1), self.semW[p])
        self.j += 1

    def finish(self):
        j = self.j - 1
        last = j & 1
        self.h_g[last].wait()
        if self.h_w[1 - last] is not None:
            self.h_w[1 - last].wait()
        pltpu.sync_copy(self.R[last], self._dst(j))


@functools.lru_cache(maxsize=None)
def _make_sc_gather(M, Ds, Cs, folds):
    n = len(Ds)
    mpw = M // NW
    ns = [mpw // C for C in Cs]
    max_n = max(ns)
    mesh = plsc.VectorSubcoreMesh(core_axis_name="c", subcore_axis_name="s")

    out_type = [
        jax.ShapeDtypeStruct((M // 2, 2 * D) if f else (M, D), F32)
        for D, f in zip(Ds, folds)]
    scratch = []
    for D, C in zip(Ds, Cs):
        scratch += [pltpu.VMEM((C,), jnp.int32), pltpu.VMEM((C,), jnp.int32),
                    pltpu.VMEM((C, D), F32), pltpu.VMEM((C, D), F32),
                    pltpu.SemaphoreType.DMA, pltpu.SemaphoreType.DMA,
                    pltpu.SemaphoreType.DMA, pltpu.SemaphoreType.DMA]

    @functools.partial(
        pl.kernel, out_type=out_type, mesh=mesh, scratch_types=scratch,
        compiler_params=pltpu.CompilerParams(use_tc_tiling_on_sc=False))
    def k(*refs):
        tables = refs[:n]
        idxs = refs[n:2 * n]
        outs = refs[2 * n:3 * n]
        scr = refs[3 * n:]
        wid = lax.axis_index("s") * NC + lax.axis_index("c")
        base0 = wid * mpw
        h = wid // (NW // 2)
        rowf = base0 - h * (M // 2)

        streams = []
        for k_ in range(n):
            sc = scr[8 * k_:8 * (k_ + 1)]
            hsl = pl.ds(h * Ds[k_], Ds[k_]) if folds[k_] else None
            row0 = rowf if folds[k_] else base0
            streams.append(_GatherStream(
                tables[k_], idxs[k_], outs[k_],
                (sc[0], sc[1]), (sc[2], sc[3]), (sc[4], sc[5]),
                (sc[6], sc[7]), base0, row0, hsl, Cs[k_]))
        for t in range(max_n):
            for k_, st in enumerate(streams):
                if (t + 1) * ns[k_] // max_n > t * ns[k_] // max_n:
                    st.step()
        for st in streams:
            st.finish()

    return k


def _sc_gather(tables, idxs, Cs, folds=None):
    M = idxs[0].shape[0]
    Ds = tuple(t.shape[1] for t in tables)
    folds = tuple(folds) if folds else tuple(False for _ in Ds)
    k = _make_sc_gather(M, Ds, tuple(Cs), folds)
    return k(*tables, *idxs)


def _scatter_chunk_loop(idx_hbm, acc, in_slices, I, Bs, semIn, semAdd,
                        tbase, C, n):
    na = len(acc)
    hin = [[None, None] for _ in range(na)]
    hadd = [[None, None] for _ in range(na)]
    for j in range(n):
        q = j & 1
        p = 1 - q
        for a in range(na):
            if hadd[a][q] is not None:
                hadd[a][q].wait()
        base = tbase + j * C
        pltpu.sync_copy(idx_hbm.at[pl.ds(base, C)], I[q])
        srcs = in_slices(j)
        for a in range(na):
            hin[a][q] = pltpu.async_copy(srcs[a], Bs[a][q], semIn[a][q])
        if j >= 1:
            for a in range(na):
                hin[a][p].wait()
                hadd[a][p] = pltpu.async_copy(
                    Bs[a][p], acc[a].at[I[p]], semAdd[a][p], add=True)
    last = (n - 1) & 1
    for a in range(na):
        hin[a][last].wait()
        hadd[a][last] = pltpu.async_copy(
            Bs[a][last], acc[a].at[I[last]], semAdd[a][last], add=True)
    for a in range(na):
        if hadd[a][1 - last] is not None:
            hadd[a][1 - last].wait()
        hadd[a][last].wait()


@functools.lru_cache(maxsize=None)
def _make_sc_scatter(M, S, C, fold_out):
    rows_per_tile = M // NS
    nchunks = rows_per_tile // C
    RS = S // NS
    mesh = plsc.VectorSubcoreMesh(core_axis_name="c", subcore_axis_name="s")

    out_shape = (S // 2, 128) if fold_out else (S, 64)

    @functools.partial(
        pl.kernel,
        out_type=jax.ShapeDtypeStruct(out_shape, F32),
        mesh=mesh,
        scratch_types=[
            pltpu.VMEM_SHARED((S, 8), F32),
            pltpu.VMEM((C,), jnp.int32), pltpu.VMEM((C,), jnp.int32),
            pltpu.VMEM((C, 8), F32), pltpu.VMEM((C, 8), F32),
            pltpu.SemaphoreType.DMA, pltpu.SemaphoreType.DMA,
            pltpu.SemaphoreType.DMA, pltpu.SemaphoreType.DMA,
        ],
        compiler_params=pltpu.CompilerParams(use_tc_tiling_on_sc=False),
    )
    def k(data_hbm, idx_hbm, zeros_hbm, out_hbm, acc_sh, I0, I1, B0, B1,
          sI0, sI1, sA0, sA1):
        c = lax.axis_index("c")
        s = lax.axis_index("s")
        tbase = s * rows_per_tile
        hin = s // (NS // 2)
        rowin = tbase - hin * (M // 2)

        for r in range(4):
            g = c * 4 + r
            pltpu.sync_copy(zeros_hbm.at[pl.ds(s * RS, RS)],
                            acc_sh.at[pl.ds(s * RS, RS)])
            plsc.subcore_barrier()
            _scatter_chunk_loop(
                idx_hbm, [acc_sh],
                lambda j: [data_hbm.at[pl.ds(rowin + j * C, C),
                                       pl.ds((hin * 8 + g) * 8, 8)]],
                (I0, I1), [(B0, B1)], [(sI0, sI1)], [(sA0, sA1)],
                tbase, C, nchunks)
            plsc.subcore_barrier()
            if fold_out:
                hout = s // (NS // 2)
                rowout = s * RS - hout * (S // 2)
                pltpu.sync_copy(
                    acc_sh.at[pl.ds(s * RS, RS)],
                    out_hbm.at[pl.ds(rowout, RS),
                               pl.ds((hout * 8 + g) * 8, 8)])
            else:
                pltpu.sync_copy(
                    acc_sh.at[pl.ds(s * RS, RS)],
                    out_hbm.at[pl.ds(s * RS, RS), pl.ds(g * 8, 8)])

    return k


def _sc_segment_sum_folded(dataf, idx, S, C):
    M2 = dataf.shape[0]
    zeros = jnp.zeros((S, 8), F32)
    k = _make_sc_scatter(2 * M2, S, C, True)
    return k(dataf, idx, zeros)


def _sc_segment_sum_plain(dataf, idx, S, C):
    M2 = dataf.shape[0]
    zeros = jnp.zeros((S, 8), F32)
    k = _make_sc_scatter(2 * M2, S, C, False)
    return k(dataf, idx, zeros)


@functools.lru_cache(maxsize=None)
def _make_sc_scatter3(M, S, C):
    rows_per_tile = M // NS
    nchunks = rows_per_tile // C
    RS = S // NS
    mesh = plsc.VectorSubcoreMesh(core_axis_name="c", subcore_axis_name="s")

    scratch = [pltpu.VMEM_SHARED((S, 32), F32) for _ in range(3)]
    scratch += [pltpu.VMEM((C,), jnp.int32), pltpu.VMEM((C,), jnp.int32)]
    for _ in range(3):
        scratch += [pltpu.VMEM((C, 32), F32), pltpu.VMEM((C, 32), F32),
                    pltpu.SemaphoreType.DMA, pltpu.SemaphoreType.DMA,
                    pltpu.SemaphoreType.DMA, pltpu.SemaphoreType.DMA]

    @functools.partial(
        pl.kernel,
        out_type=[jax.ShapeDtypeStruct((S, 64), F32) for _ in range(3)],
        mesh=mesh,
        scratch_types=scratch,
        compiler_params=pltpu.CompilerParams(use_tc_tiling_on_sc=False),
    )
    def k(d0, d1, d2, idx_hbm, zeros_hbm, o0, o1, o2, a0, a1, a2, I0, I1,
          *bs):
        c = lax.axis_index("c")
        s = lax.axis_index("s")
        tbase = s * rows_per_tile
        hin = s // (NS // 2)
        rowin = tbase - hin * (M // 2)
        acc = [a0, a1, a2]
        datas = [d0, d1, d2]
        outs = [o0, o1, o2]
        Bs = [(bs[6 * a], bs[6 * a + 1]) for a in range(3)]
        semIn = [(bs[6 * a + 2], bs[6 * a + 3]) for a in range(3)]
        semAdd = [(bs[6 * a + 4], bs[6 * a + 5]) for a in range(3)]

        for a in range(3):
            pltpu.sync_copy(zeros_hbm.at[pl.ds(s * RS, RS)],
                            acc[a].at[pl.ds(s * RS, RS)])
        plsc.subcore_barrier()
        _scatter_chunk_loop(
            idx_hbm, acc,
            lambda j: [d.at[pl.ds(rowin + j * C, C),
                            pl.ds((hin * 8 + c * 4) * 8, 32)]
                       for d in datas],
            (I0, I1), Bs, semIn, semAdd, tbase, C, nchunks)
        plsc.subcore_barrier()
        for a in range(3):
            pltpu.sync_copy(
                acc[a].at[pl.ds(s * RS, RS)],
                outs[a].at[pl.ds(s * RS, RS), pl.ds(c * 32, 32)])

    return k


def _sc_segment_sum3(datafs, idx, S, C):
    M2 = datafs[0].shape[0]
    zeros = jnp.zeros((S, 32), F32)
    k = _make_sc_scatter3(2 * M2, S, C)
    return list(k(datafs[0], datafs[1], datafs[2], idx, zeros))


BH = 1600


def _tc_emb_tables(emb_table_p, W_src, W_dst):
    def body(et, ws, wd, a_ref, b_ref):
        a_ref[...] = jnp.dot(et[...], ws[...], preferred_element_type=F32)
        b_ref[...] = jnp.dot(et[...], wd[...], preferred_element_type=F32)

    return pl.pallas_call(
        body,
        out_shape=[jax.ShapeDtypeStruct((96, EMB), F32),
                   jax.ShapeDtypeStruct((96, EMB), F32)],
    )(emb_table_p, W_src, W_dst)


def _tc_edge(gA, gB, psd, freq2, W3, b_emb, Wro0):
    E2 = gA.shape[0]
    grid = (E2 // BH,)

    def body(gA_r, gB_r, psd_r, fq_r, W3_r, b_r, Wro_r, m_r, geo2_r, g0_r):
        ms, gs, g0s = [], [], []
        for h in range(2):
            ps = psd_r[:, 8 * h:8 * h + 3]
            pd = psd_r[:, 16 + 8 * h:16 + 8 * h + 3]
            rel = pd - ps
            d2 = jnp.sum(rel * rel, axis=1, keepdims=True)
            d = jnp.sqrt(d2 + 1e-9)
            x = d / CUT
            x5 = x * x * x * x * x
            env = 1.0 - 21.0 * x5 + 35.0 * x5 * x - 15.0 * x5 * x * x
            env = jnp.where(x < 1.0, env, 0.0)
            sinval = jnp.sin(fq_r[...] * x)
            rbf = jnp.float32((2.0 / CUT) ** 0.5) * env * sinval / d
            m = _swish(gA_r[:, 64 * h:64 * h + 64]
                       + gB_r[:, 64 * h:64 * h + 64]
                       + jnp.dot(rbf, W3_r[...], preferred_element_type=F32)
                       + b_r[...])
            ms.append(m)
            gs.append(jnp.concatenate(
                [rel, rbf, jnp.zeros((rel.shape[0], 7), F32)], axis=1))
            g0s.append(
                jnp.dot(rbf, Wro_r[...], preferred_element_type=F32) * m)
        m_r[...] = jnp.concatenate(ms, axis=1)
        geo2_r[...] = jnp.concatenate(gs, axis=1)
        g0_r[...] = jnp.concatenate(g0s, axis=1)

    bspec = lambda D: pl.BlockSpec((BH, D), lambda i: (i, 0))
    wspec = lambda s: pl.BlockSpec(s, lambda i: tuple(0 for _ in s))
    return pl.pallas_call(
        body,
        grid=grid,
        in_specs=[bspec(128), bspec(128), bspec(32),
                  wspec((1, NRAD)), wspec((NRAD, EMB)), wspec((1, EMB)),
                  wspec((NRAD, EMB))],
        out_specs=[bspec(128), bspec(32), bspec(128)],
        out_shape=[jax.ShapeDtypeStruct((E2, 128), F32),
                   jax.ShapeDtypeStruct((E2, 32), F32),
                   jax.ShapeDtypeStruct((E2, 128), F32)],
    )(gA, gB, psd, freq2, W3, b_emb, Wro0)


def _tc_sbfw(gkj, gji, Wcat):
    T2 = gkj.shape[0]
    grid = (T2 // BH,)

    def body(kj_r, ji_r, W_r, out_r):
        outs = []
        for h in range(2):
            kjh = kj_r[:, 16 * h:16 * h + 16]
            jih = ji_r[:, 16 * h:16 * h + 16]
            R1 = kjh[:, 0:3]
            R2 = jih[:, 0:3]
            rbfk = kjh[:, 3:9]
            dotp = jnp.sum(R1 * R2, axis=1, keepdims=True)
            cx = R1[:, 1:2] * R2[:, 2:3] - R1[:, 2:3] * R2[:, 1:2]
            cy = R1[:, 2:3] * R2[:, 0:1] - R1[:, 0:1] * R2[:, 2:3]
            cz = R1[:, 0:1] * R2[:, 1:2] - R1[:, 1:2] * R2[:, 0:1]
            c2 = cx * cx + cy * cy + cz * cz
            nrm = jnp.sqrt(dotp * dotp + c2)
            ct = jnp.where(nrm > 0.0,
                           dotp / jnp.where(nrm > 0.0, nrm, 1.0), 1.0)
            cheb = [jnp.ones_like(ct), ct]
            for l in range(2, NSPH):
                cheb.append(2.0 * ct * cheb[-1] - cheb[-2])
            F = jnp.concatenate([cb * rbfk for cb in cheb], axis=1)
            outs.append(jnp.dot(F, W_r[...], preferred_element_type=F32))
        out_r[...] = jnp.concatenate(outs, axis=1)

    bspec = lambda D: pl.BlockSpec((BH, D), lambda i: (i, 0))
    return pl.pallas_call(
        body,
        grid=grid,
        in_specs=[bspec(32), bspec(32),
                  pl.BlockSpec((NSPH * NRAD, 16), lambda i: (0, 0))],
        out_specs=bspec(32),
        out_shape=jax.ShapeDtypeStruct((T2, 32), F32),
    )(gkj, gji, Wcat)


def _tc_pre(m, geo2, Wji, bji, Wkj, bkj, Wri):
    E2 = m.shape[0]
    grid = (E2 // BH,)

    def body(m_r, geo2_r, Wji_r, bji_r, Wkj_r, bkj_r, Wri_r,
             xji_r, xkj_r):
        xjis, xkjs = [], []
        for h in range(2):
            mm = m_r[:, 64 * h:64 * h + 64]
            rbf = geo2_r[:, 16 * h + 3:16 * h + 9]
            xjis.append(_swish(
                jnp.dot(mm, Wji_r[...], preferred_element_type=F32)
                + bji_r[...]))
            rbfw = jnp.dot(rbf, Wri_r[...], preferred_element_type=F32)
            xkjs.append(_swish(
                jnp.dot(mm, Wkj_r[...], preferred_element_type=F32)
                + bkj_r[...]) * rbfw)
        xji_r[...] = jnp.concatenate(xjis, axis=1)
        xkj_r[...] = jnp.concatenate(xkjs, axis=1)

    bspec = lambda D: pl.BlockSpec((BH, D), lambda i: (i, 0))
    wspec = lambda s: pl.BlockSpec(s, lambda i: tuple(0 for _ in s))
    return pl.pallas_call(
        body,
        grid=grid,
        in_specs=[bspec(128), bspec(32),
                  wspec((EMB, EMB)), wspec((1, EMB)),
                  wspec((EMB, EMB)), wspec((1, EMB)), wspec((NRAD, EMB))],
        out_specs=[bspec(128), bspec(128)],
        out_shape=[jax.ShapeDtypeStruct((E2, 128), F32),
                   jax.ShapeDtypeStruct((E2, 128), F32)],
    )(m, geo2, Wji, bji, Wkj, bkj, Wri)


def _tc_bilinear(xk, sbfw, Wb, col0):
    T2 = xk.shape[0]
    grid = (T2 // BH,)

    def body(xk_r, sb_r, Wb_r, out_r):
        outs = []
        for h in range(2):
            xkk = xk_r[:, 64 * h:64 * h + 64]
            acc = jnp.zeros((xkk.shape[0], EMB), F32)
            for l in range(NBIL):
                c = 16 * h + col0 + l
                acc = acc + jnp.dot(xkk, Wb_r[l],
                                    preferred_element_type=F32) \
                    * sb_r[:, c:c + 1]
            outs.append(acc)
        out_r[...] = jnp.concatenate(outs, axis=1)

    bspec = lambda D: pl.BlockSpec((BH, D), lambda i: (i, 0))
    return pl.pallas_call(
        body,
        grid=grid,
        in_specs=[bspec(128), bspec(32),
                  pl.BlockSpec((NBIL, EMB, EMB), lambda i: (0, 0, 0))],
        out_specs=bspec(128),
        out_shape=jax.ShapeDtypeStruct((T2, 128), F32),
    )(xk, sbfw, Wb)


def _tc_post(xji, agg, m, geo2, Wr1, br1, Wsk, bsk, Wr2, br2, Wro):
    E2 = m.shape[0]
    grid = (E2 // BH,)

    def body(xji_r, agg_r, m_r, geo2_r, Wr1_r, br1_r, Wsk_r, bsk_r,
             Wr2_r, br2_r, Wro_r, mout_r, g_r):
        mouts, gouts = [], []
        for h in range(2):
            sl = slice(64 * h, 64 * h + 64)
            hh = xji_r[:, sl] + agg_r[:, sl]
            for rr in range(NBEF):
                tmp = _swish(jnp.dot(hh, Wr1_r[2 * rr],
                                     preferred_element_type=F32)
                             + br1_r[2 * rr:2 * rr + 1, 0, :])
                tmp = _swish(jnp.dot(tmp, Wr1_r[2 * rr + 1],
                                     preferred_element_type=F32)
                             + br1_r[2 * rr + 1:2 * rr + 2, 0, :])
                hh = hh + tmp
            mm = _swish(jnp.dot(hh, Wsk_r[...], preferred_element_type=F32)
                        + bsk_r[...]) + m_r[:, sl]
            for rr in range(NAFT):
                tmp = _swish(jnp.dot(mm, Wr2_r[2 * rr],
                                     preferred_element_type=F32)
                             + br2_r[2 * rr:2 * rr + 1, 0, :])
                tmp = _swish(jnp.dot(tmp, Wr2_r[2 * rr + 1],
                                     preferred_element_type=F32)
                             + br2_r[2 * rr + 1:2 * rr + 2, 0, :])
                mm = mm + tmp
            mouts.append(mm)
            rbf = geo2_r[:, 16 * h + 3:16 * h + 9]
            gouts.append(
                jnp.dot(rbf, Wro_r[...], preferred_element_type=F32) * mm)
        mout_r[...] = jnp.concatenate(mouts, axis=1)
        g_r[...] = jnp.concatenate(gouts, axis=1)

    bspec = lambda D: pl.BlockSpec((BH, D), lambda i: (i, 0))
    wspec = lambda s: pl.BlockSpec(s, lambda i: tuple(0 for _ in s))
    return pl.pallas_call(
        body,
        grid=grid,
        in_specs=[bspec(128), bspec(128), bspec(128), bspec(32),
                  wspec((2 * NBEF, EMB, EMB)), wspec((2 * NBEF, 1, EMB)),
                  wspec((EMB, EMB)), wspec((1, EMB)),
                  wspec((2 * NAFT, EMB, EMB)), wspec((2 * NAFT, 1, EMB)),
                  wspec((NRAD, EMB))],
        out_specs=[bspec(128), bspec(128)],
        out_shape=[jax.ShapeDtypeStruct((E2, 128), F32),
                   jax.ShapeDtypeStruct((E2, 128), F32)],
    )(xji, agg, m, geo2, Wr1, br1, Wsk, bsk, Wr2, br2, Wro)


def _tc_out(t0, t1, t2, Wd, bd, Wo, BN):
    N = t0.shape[0]
    grid = (N // BN,)

    def body(t0_r, t1_r, t2_r, Wd_r, bd_r, Wo_r, P_r):
        acc = jnp.zeros((t0_r.shape[0], NT), F32)
        for i, t_r in enumerate((t0_r, t1_r, t2_r)):
            t = t_r[...]
            for j in range(NDENSE):
                t = _swish(jnp.dot(t, Wd_r[i, j],
                                   preferred_element_type=F32)
                           + bd_r[i, j:j + 1, :].reshape(1, EMB))
            acc = acc + jnp.dot(t, Wo_r[i], preferred_element_type=F32)
        P_r[...] = acc

    bspec = lambda D: pl.BlockSpec((BN, D), lambda i: (i, 0))
    wspec = lambda s: pl.BlockSpec(s, lambda i: tuple(0 for _ in s))
    return pl.pallas_call(
        body,
        grid=grid,
        in_specs=[bspec(EMB), bspec(EMB), bspec(EMB),
                  wspec((NB + 1, NDENSE, EMB, EMB)),
                  wspec((NB + 1, NDENSE, EMB)),
                  wspec((NB + 1, EMB, NT))],
        out_specs=bspec(NT),
        out_shape=jax.ShapeDtypeStruct((N, NT), F32),
    )(t0, t1, t2, Wd, bd, Wo)


def kernel(pos, freq, emb_table, W_emb, b_emb, W_rbf_int, W_sbf_int, W_ji,
           b_ji, W_kj, b_kj, W_bilin, W_res1, b_res1, W_skip, b_skip, W_res2,
           b_res2, W_rbf_out, W_dense_out, b_dense_out, W_out, Z, edge_index,
           triplet_kj, triplet_ji):
    N = pos.shape[0]
    E = edge_index.shape[1]
    T = triplet_kj.shape[0]
    BN = 2000

    src = edge_index[0].astype(jnp.int32)
    dst = edge_index[1].astype(jnp.int32)
    kj = triplet_kj.astype(jnp.int32)
    ji = triplet_ji.astype(jnp.int32)
    kj2 = jnp.where(kj < E // 2, 2 * kj, 2 * (kj - E // 2) + 1)
    ji2 = jnp.where(ji < E // 2, 2 * ji, 2 * (ji - E // 2) + 1)

    NPAD = 10240
    Zp = jnp.pad(Z.astype(jnp.int32), (0, NPAD - N))
    posP = jnp.pad(pos.astype(F32), ((0, NPAD - N), (0, 8 - 3)))
    emb_p = jnp.pad(emb_table.astype(F32), ((0, 96 - 95), (0, 0)))
    W_src = W_emb[:EMB]
    W_dst = W_emb[EMB:2 * EMB]
    W3 = W_emb[2 * EMB:]
    b_emb2 = b_emb.reshape(1, EMB)
    Wcat = jnp.concatenate([W_sbf_int[0], W_sbf_int[1]], axis=1)
    Wb_t = jnp.transpose(W_bilin, (0, 2, 1, 3))
    b_ji2 = b_ji.reshape(NB, 1, EMB)
    b_kj2 = b_kj.reshape(NB, 1, EMB)
    b_res1_2 = b_res1.reshape(NB, 2 * NBEF, 1, EMB)
    b_skip2 = b_skip.reshape(NB, 1, EMB)
    b_res2_2 = b_res2.reshape(NB, 2 * NAFT, 1, EMB)

    A_tab, B_tab = _tc_emb_tables(emb_p, W_src, W_dst)

    nodeA, nodeB = _sc_gather((A_tab, B_tab), (Zp, Zp), (320, 320))

    gA, gB, ps, pd = _sc_gather(
        (nodeA, nodeB, posP, posP), (src, dst, src, dst),
        (200, 200, 1000, 1000), folds=(True, True, True, True))
    gA = gA.reshape(E // 2, 128)
    gB = gB.reshape(E // 2, 128)
    psd = jnp.concatenate(
        [ps.reshape(E // 2, 16), pd.reshape(E // 2, 16)], axis=1)

    m, geo2, g0 = _tc_edge(gA, gB, psd,
                           freq.reshape(1, NRAD).astype(F32),
                           W3, b_emb2, W_rbf_out[0])
    geom = geo2.reshape(E, 16)

    gkj, gji = _sc_gather((geom, geom), (kj2, ji2), (1000, 1000),
                          folds=(True, True))

    sbfw = _tc_sbfw(gkj, gji, Wcat)

    ts = [_sc_segment_sum_plain(g0, dst, NPAD, C=2000)]
    for i in range(NB):
        xji, xkj = _tc_pre(m, geo2, W_ji[i], b_ji2[i], W_kj[i],
                           b_kj2[i], W_rbf_int[i])
        (xk,) = _sc_gather((xkj.reshape(E, 64),), (kj2,), (200,),
                           folds=(True,))
        xk = xk.reshape(T // 2, 128)
        xt = _tc_bilinear(xk, sbfw, Wb_t[i], NBIL * i)
        agg = _sc_segment_sum_folded(xt, ji, E, C=2000)
        m, g = _tc_post(xji, agg, m, geo2, W_res1[i], b_res1_2[i],
                        W_skip[i], b_skip2[i], W_res2[i], b_res2_2[i],
                        W_rbf_out[i + 1])
        ts.append(_sc_segment_sum_plain(g, dst, NPAD, C=2000))

    t0, t1, t2 = (t[:N] for t in ts)

    return _tc_out(t0, t1, t2, W_dense_out, b_dense_out, W_out, BN)


# speedup vs baseline: 1.0013x; 1.0013x over previous
import functools

import jax
import jax.numpy as jnp
from jax import lax
from jax.experimental import pallas as pl
from jax.experimental.pallas import tpu as pltpu
from jax.experimental.pallas import tpu_sc as plsc

EMB = 64
NB = 2
NBIL = 8
NSPH = 7
NRAD = 6
CUT = 5.0
NBEF = 1
NAFT = 2
NDENSE = 3
NT = 12

NC = 2
NS = 16
NW = NC * NS

F32 = jnp.float32


def _swish(x):
    return x * jax.nn.sigmoid(x)


class _GatherStream:
    def __init__(self, tab, idx, out, I, R, semG, semW, base0, row0, hsl, C):
        self.tab, self.idx, self.out = tab, idx, out
        self.I, self.R, self.semG, self.semW = I, R, semG, semW
        self.base0, self.row0, self.hsl, self.C = base0, row0, hsl, C
        self.h_g = [None, None]
        self.h_w = [None, None]
        self.j = 0

    def _dst(self, j):
        r = self.row0 + j * self.C
        if self.hsl is None:
            return self.out.at[pl.ds(r, self.C)]
        return self.out.at[pl.ds(r, self.C), self.hsl]

    def step(self):
        j = self.j
        q = j & 1
        p = 1 - q
        if self.h_w[q] is not None:
            self.h_w[q].wait()
        pltpu.sync_copy(
            self.idx.at[pl.ds(self.base0 + j * self.C, self.C)], self.I[q])
        self.h_g[q] = pltpu.async_copy(
            self.tab.at[self.I[q]], self.R[q], self.semG[q])
        if j >= 1:
            self.h_g[p].wait()
            self.h_w[p] = pltpu.async_copy(
                self.R[p], self._dst(j - 1), self.semW[p])
        self.j += 1

    def finish(self):
        j = self.j - 1
        last = j & 1
        self.h_g[last].wait()
        if self.h_w[1 - last] is not None:
            self.h_w[1 - last].wait()
        pltpu.sync_copy(self.R[last], self._dst(j))


@functools.lru_cache(maxsize=None)
def _make_sc_gather(M, Ds, Cs, folds):
    n = len(Ds)
    mpw = M // NW
    ns = [mpw // C for C in Cs]
    max_n = max(ns)
    mesh = plsc.VectorSubcoreMesh(core_axis_name="c", subcore_axis_name="s")

    out_type = [
        jax.ShapeDtypeStruct((M // 2, 2 * D) if f else (M, D), F32)
        for D, f in zip(Ds, folds)]
    scratch = []
    for D, C in zip(Ds, Cs):
        scratch += [pltpu.VMEM((C,), jnp.int32), pltpu.VMEM((C,), jnp.int32),
                    pltpu.VMEM((C, D), F32), pltpu.VMEM((C, D), F32),
                    pltpu.SemaphoreType.DMA, pltpu.SemaphoreType.DMA,
                    pltpu.SemaphoreType.DMA, pltpu.SemaphoreType.DMA]

    @functools.partial(
        pl.kernel, out_type=out_type, mesh=mesh, scratch_types=scratch,
        compiler_params=pltpu.CompilerParams(use_tc_tiling_on_sc=False))
    def k(*refs):
        tables = refs[:n]
        idxs = refs[n:2 * n]
        outs = refs[2 * n:3 * n]
        scr = refs[3 * n:]
        wid = lax.axis_index("s") * NC + lax.axis_index("c")
        base0 = wid * mpw
        h = wid // (NW // 2)
        rowf = base0 - h * (M // 2)

        streams = []
        for k_ in range(n):
            sc = scr[8 * k_:8 * (k_ + 1)]
            hsl = pl.ds(h * Ds[k_], Ds[k_]) if folds[k_] else None
            row0 = rowf if folds[k_] else base0
            streams.append(_GatherStream(
                tables[k_], idxs[k_], outs[k_],
                (sc[0], sc[1]), (sc[2], sc[3]), (sc[4], sc[5]),
                (sc[6], sc[7]), base0, row0, hsl, Cs[k_]))
        for t in range(max_n):
            for k_, st in enumerate(streams):
                if (t + 1) * ns[k_] // max_n > t * ns[k_] // max_n:
                    st.step()
        for st in streams:
            st.finish()

    return k


def _sc_gather(tables, idxs, Cs, folds=None):
    M = idxs[0].shape[0]
    Ds = tuple(t.shape[1] for t in tables)
    folds = tuple(folds) if folds else tuple(False for _ in Ds)
    k = _make_sc_gather(M, Ds, tuple(Cs), folds)
    return k(*tables, *idxs)


def _scatter_chunk_loop(idx_hbm, acc, in_slices, I, Bs, semIn, semAdd,
                        tbase, C, n):
    na = len(acc)
    hin = [[None, None] for _ in range(na)]
    hadd = [[None, None] for _ in range(na)]
    for j in range(n):
        q = j & 1
        p = 1 - q
        for a in range(na):
            if hadd[a][q] is not None:
                hadd[a][q].wait()
        base = tbase + j * C
        pltpu.sync_copy(idx_hbm.at[pl.ds(base, C)], I[q])
        srcs = in_slices(j)
        for a in range(na):
            hin[a][q] = pltpu.async_copy(srcs[a], Bs[a][q], semIn[a][q])
        if j >= 1:
            for a in range(na):
                hin[a][p].wait()
                hadd[a][p] = pltpu.async_copy(
                    Bs[a][p], acc[a].at[I[p]], semAdd[a][p], add=True)
    last = (n - 1) & 1
    for a in range(na):
        hin[a][last].wait()
        hadd[a][last] = pltpu.async_copy(
            Bs[a][last], acc[a].at[I[last]], semAdd[a][last], add=True)
    for a in range(na):
        if hadd[a][1 - last] is not None:
            hadd[a][1 - last].wait()
        hadd[a][last].wait()


@functools.lru_cache(maxsize=None)
def _make_sc_scatter(M, S, C, fold_out):
    rows_per_tile = M // NS
    nchunks = rows_per_tile // C
    RS = S // NS
    mesh = plsc.VectorSubcoreMesh(core_axis_name="c", subcore_axis_name="s")

    out_shape = (S // 2, 128) if fold_out else (S, 64)

    @functools.partial(
        pl.kernel,
        out_type=jax.ShapeDtypeStruct(out_shape, F32),
        mesh=mesh,
        scratch_types=[
            pltpu.VMEM_SHARED((S, 8), F32),
            pltpu.VMEM((C,), jnp.int32), pltpu.VMEM((C,), jnp.int32),
            pltpu.VMEM((C, 8), F32), pltpu.VMEM((C, 8), F32),
            pltpu.SemaphoreType.DMA, pltpu.SemaphoreType.DMA,
            pltpu.SemaphoreType.DMA, pltpu.SemaphoreType.DMA,
        ],
        compiler_params=pltpu.CompilerParams(use_tc_tiling_on_sc=False),
    )
    def k(data_hbm, idx_hbm, zeros_hbm, out_hbm, acc_sh, I0, I1, B0, B1,
          sI0, sI1, sA0, sA1):
        c = lax.axis_index("c")
        s = lax.axis_index("s")
        tbase = s * rows_per_tile
        hin = s // (NS // 2)
        rowin = tbase - hin * (M // 2)

        for r in range(4):
            g = c * 4 + r
            pltpu.sync_copy(zeros_hbm.at[pl.ds(s * RS, RS)],
                            acc_sh.at[pl.ds(s * RS, RS)])
            plsc.subcore_barrier()
            _scatter_chunk_loop(
                idx_hbm, [acc_sh],
                lambda j: [data_hbm.at[pl.ds(rowin + j * C, C),
                                       pl.ds((hin * 8 + g) * 8, 8)]],
                (I0, I1), [(B0, B1)], [(sI0, sI1)], [(sA0, sA1)],
                tbase, C, nchunks)
            plsc.subcore_barrier()
            if fold_out:
                hout = s // (NS // 2)
                rowout = s * RS - hout * (S // 2)
                pltpu.sync_copy(
                    acc_sh.at[pl.ds(s * RS, RS)],
                    out_hbm.at[pl.ds(rowout, RS),
                               pl.ds((hout * 8 + g) * 8, 8)])
            else:
                pltpu.sync_copy(
                    acc_sh.at[pl.ds(s * RS, RS)],
                    out_hbm.at[pl.ds(s * RS, RS), pl.ds(g * 8, 8)])

    return k


def _sc_segment_sum_folded(dataf, idx, S, C):
    M2 = dataf.shape[0]
    zeros = jnp.zeros((S, 8), F32)
    k = _make_sc_scatter(2 * M2, S, C, True)
    return k(dataf, idx, zeros)


def _sc_segment_sum_plain(dataf, idx, S, C):
    M2 = dataf.shape[0]
    zeros = jnp.zeros((S, 8), F32)
    k = _make_sc_scatter(2 * M2, S, C, False)
    return k(dataf, idx, zeros)


@functools.lru_cache(maxsize=None)
def _make_sc_scatter3(M, S, C):
    rows_per_tile = M // NS
    nchunks = rows_per_tile // C
    RS = S // NS
    mesh = plsc.VectorSubcoreMesh(core_axis_name="c", subcore_axis_name="s")

    scratch = [pltpu.VMEM_SHARED((S, 32), F32) for _ in range(3)]
    scratch += [pltpu.VMEM((C,), jnp.int32), pltpu.VMEM((C,), jnp.int32)]
    for _ in range(3):
        scratch += [pltpu.VMEM((C, 32), F32), pltpu.VMEM((C, 32), F32),
                    pltpu.SemaphoreType.DMA, pltpu.SemaphoreType.DMA,
                    pltpu.SemaphoreType.DMA, pltpu.SemaphoreType.DMA]

    @functools.partial(
        pl.kernel,
        out_type=[jax.ShapeDtypeStruct((S, 64), F32) for _ in range(3)],
        mesh=mesh,
        scratch_types=scratch,
        compiler_params=pltpu.CompilerParams(use_tc_tiling_on_sc=False),
    )
    def k(d0, d1, d2, idx_hbm, zeros_hbm, o0, o1, o2, a0, a1, a2, I0, I1,
          *bs):
        c = lax.axis_index("c")
        s = lax.axis_index("s")
        tbase = s * rows_per_tile
        hin = s // (NS // 2)
        rowin = tbase - hin * (M // 2)
        acc = [a0, a1, a2]
        datas = [d0, d1, d2]
        outs = [o0, o1, o2]
        Bs = [(bs[6 * a], bs[6 * a + 1]) for a in range(3)]
        semIn = [(bs[6 * a + 2], bs[6 * a + 3]) for a in range(3)]
        semAdd = [(bs[6 * a + 4], bs[6 * a + 5]) for a in range(3)]

        for a in range(3):
            pltpu.sync_copy(zeros_hbm.at[pl.ds(s * RS, RS)],
                            acc[a].at[pl.ds(s * RS, RS)])
        plsc.subcore_barrier()
        _scatter_chunk_loop(
            idx_hbm, acc,
            lambda j: [d.at[pl.ds(rowin + j * C, C),
                            pl.ds((hin * 8 + c * 4) * 8, 32)]
                       for d in datas],
            (I0, I1), Bs, semIn, semAdd, tbase, C, nchunks)
        plsc.subcore_barrier()
        for a in range(3):
            pltpu.sync_copy(
                acc[a].at[pl.ds(s * RS, RS)],
                outs[a].at[pl.ds(s * RS, RS), pl.ds(c * 32, 32)])

    return k


def _sc_segment_sum3(datafs, idx, S, C):
    M2 = datafs[0].shape[0]
    zeros = jnp.zeros((S, 32), F32)
    k = _make_sc_scatter3(2 * M2, S, C)
    return list(k(datafs[0], datafs[1], datafs[2], idx, zeros))


BH = 1600


def _tc_emb_tables(emb_table_p, W_src, W_dst):
    def body(et, ws, wd, a_ref, b_ref):
        a_ref[...] = jnp.dot(et[...], ws[...], preferred_element_type=F32)
        b_ref[...] = jnp.dot(et[...], wd[...], preferred_element_type=F32)

    return pl.pallas_call(
        body,
        out_shape=[jax.ShapeDtypeStruct((96, EMB), F32),
                   jax.ShapeDtypeStruct((96, EMB), F32)],
    )(emb_table_p, W_src, W_dst)


def _tc_edge(gA, gB, psd, freq2, W3, b_emb, Wro0):
    E2 = gA.shape[0]
    grid = (E2 // BH,)

    def body(gA_r, gB_r, psd_r, fq_r, W3_r, b_r, Wro_r, m_r, geo2_r, g0_r):
        ms, gs, g0s = [], [], []
        for h in range(2):
            ps = psd_r[:, 8 * h:8 * h + 3]
            pd = psd_r[:, 16 + 8 * h:16 + 8 * h + 3]
            rel = pd - ps
            d2 = jnp.sum(rel * rel, axis=1, keepdims=True)
            d = jnp.sqrt(d2 + 1e-9)
            x = d / CUT
            x5 = x * x * x * x * x
            env = 1.0 - 21.0 * x5 + 35.0 * x5 * x - 15.0 * x5 * x * x
            env = jnp.where(x < 1.0, env, 0.0)
            sinval = jnp.sin(fq_r[...] * x)
            rbf = jnp.float32((2.0 / CUT) ** 0.5) * env * sinval / d
            m = _swish(gA_r[:, 64 * h:64 * h + 64]
                       + gB_r[:, 64 * h:64 * h + 64]
                       + jnp.dot(rbf, W3_r[...], preferred_element_type=F32)
                       + b_r[...])
            ms.append(m)
            gs.append(jnp.concatenate(
                [rel, rbf, jnp.zeros((rel.shape[0], 7), F32)], axis=1))
            g0s.append(
                jnp.dot(rbf, Wro_r[...], preferred_element_type=F32) * m)
        m_r[...] = jnp.concatenate(ms, axis=1)
        geo2_r[...] = jnp.concatenate(gs, axis=1)
        g0_r[...] = jnp.concatenate(g0s, axis=1)

    bspec = lambda D: pl.BlockSpec((BH, D), lambda i: (i, 0))
    wspec = lambda s: pl.BlockSpec(s, lambda i: tuple(0 for _ in s))
    return pl.pallas_call(
        body,
        grid=grid,
        in_specs=[bspec(128), bspec(128), bspec(32),
                  wspec((1, NRAD)), wspec((NRAD, EMB)), wspec((1, EMB)),
                  wspec((NRAD, EMB))],
        out_specs=[bspec(128), bspec(32), bspec(128)],
        out_shape=[jax.ShapeDtypeStruct((E2, 128), F32),
                   jax.ShapeDtypeStruct((E2, 32), F32),
                   jax.ShapeDtypeStruct((E2, 128), F32)],
    )(gA, gB, psd, freq2, W3, b_emb, Wro0)


def _tc_sbfw(gkj, gji, Wcat):
    T2 = gkj.shape[0]
    grid = (T2 // BH,)

    def body(kj_r, ji_r, W_r, out_r):
        outs = []
        for h in range(2):
            kjh = kj_r[:, 16 * h:16 * h + 16]
            jih = ji_r[:, 16 * h:16 * h + 16]
            R1 = kjh[:, 0:3]
            R2 = jih[:, 0:3]
            rbfk = kjh[:, 3:9]
            dotp = jnp.sum(R1 * R2, axis=1, keepdims=True)
            cx = R1[:, 1:2] * R2[:, 2:3] - R1[:, 2:3] * R2[:, 1:2]
            cy = R1[:, 2:3] * R2[:, 0:1] - R1[:, 0:1] * R2[:, 2:3]
            cz = R1[:, 0:1] * R2[:, 1:2] - R1[:, 1:2] * R2[:, 0:1]
            c2 = cx * cx + cy * cy + cz * cz
            nrm = jnp.sqrt(dotp * dotp + c2)
            ct = jnp.where(nrm > 0.0,
                           dotp / jnp.where(nrm > 0.0, nrm, 1.0), 1.0)
            cheb = [jnp.ones_like(ct), ct]
            for l in range(2, NSPH):
                cheb.append(2.0 * ct * cheb[-1] - cheb[-2])
            F = jnp.concatenate([cb * rbfk for cb in cheb], axis=1)
            outs.append(jnp.dot(F, W_r[...], preferred_element_type=F32))
        out_r[...] = jnp.concatenate(outs, axis=1)

    bspec = lambda D: pl.BlockSpec((BH, D), lambda i: (i, 0))
    return pl.pallas_call(
        body,
        grid=grid,
        in_specs=[bspec(32), bspec(32),
                  pl.BlockSpec((NSPH * NRAD, 16), lambda i: (0, 0))],
        out_specs=bspec(32),
        out_shape=jax.ShapeDtypeStruct((T2, 32), F32),
    )(gkj, gji, Wcat)


def _tc_pre(m, geo2, Wji, bji, Wkj, bkj, Wri):
    E2 = m.shape[0]
    grid = (E2 // BH,)

    def body(m_r, geo2_r, Wji_r, bji_r, Wkj_r, bkj_r, Wri_r,
             xji_r, xkj_r):
        xjis, xkjs = [], []
        for h in range(2):
            mm = m_r[:, 64 * h:64 * h + 64]
            rbf = geo2_r[:, 16 * h + 3:16 * h + 9]
            xjis.append(_swish(
                jnp.dot(mm, Wji_r[...], preferred_element_type=F32)
                + bji_r[...]))
            rbfw = jnp.dot(rbf, Wri_r[...], preferred_element_type=F32)
            xkjs.append(_swish(
                jnp.dot(mm, Wkj_r[...], preferred_element_type=F32)
                + bkj_r[...]) * rbfw)
        xji_r[...] = jnp.concatenate(xjis, axis=1)
        xkj_r[...] = jnp.concatenate(xkjs, axis=1)

    bspec = lambda D: pl.BlockSpec((BH, D), lambda i: (i, 0))
    wspec = lambda s: pl.BlockSpec(s, lambda i: tuple(0 for _ in s))
    return pl.pallas_call(
        body,
        grid=grid,
        in_specs=[bspec(128), bspec(32),
                  wspec((EMB, EMB)), wspec((1, EMB)),
                  wspec((EMB, EMB)), wspec((1, EMB)), wspec((NRAD, EMB))],
        out_specs=[bspec(128), bspec(128)],
        out_shape=[jax.ShapeDtypeStruct((E2, 128), F32),
                   jax.ShapeDtypeStruct((E2, 128), F32)],
    )(m, geo2, Wji, bji, Wkj, bkj, Wri)


def _tc_bilinear(xk, sbfw, Wb, col0):
    T2 = xk.shape[0]
    grid = (T2 // BH,)

    def body(xk_r, sb_r, Wb_r, out_r):
        outs = []
        for h in range(2):
            xkk = xk_r[:, 64 * h:64 * h + 64]
            acc = jnp.zeros((xkk.shape[0], EMB), F32)
            for l in range(NBIL):
                c = 16 * h + col0 + l
                acc = acc + jnp.dot(xkk, Wb_r[l],
                                    preferred_element_type=F32) \
                    * sb_r[:, c:c + 1]
            outs.append(acc)
        out_r[...] = jnp.concatenate(outs, axis=1)

    bspec = lambda D: pl.BlockSpec((BH, D), lambda i: (i, 0))
    return pl.pallas_call(
        body,
        grid=grid,
        in_specs=[bspec(128), bspec(32),
                  pl.BlockSpec((NBIL, EMB, EMB), lambda i: (0, 0, 0))],
        out_specs=bspec(128),
        out_shape=jax.ShapeDtypeStruct((T2, 128), F32),
    )(xk, sbfw, Wb)


def _tc_post(xji, agg, m, geo2, Wr1, br1, Wsk, bsk, Wr2, br2, Wro):
    E2 = m.shape[0]
    grid = (E2 // BH,)

    def body(xji_r, agg_r, m_r, geo2_r, Wr1_r, br1_r, Wsk_r, bsk_r,
             Wr2_r, br2_r, Wro_r, mout_r, g_r):
        mouts, gouts = [], []
        for h in range(2):
            sl = slice(64 * h, 64 * h + 64)
            hh = xji_r[:, sl] + agg_r[:, sl]
            for rr in range(NBEF):
                tmp = _swish(jnp.dot(hh, Wr1_r[2 * rr],
                                     preferred_element_type=F32)
                             + br1_r[2 * rr:2 * rr + 1, 0, :])
                tmp = _swish(jnp.dot(tmp, Wr1_r[2 * rr + 1],
                                     preferred_element_type=F32)
                             + br1_r[2 * rr + 1:2 * rr + 2, 0, :])
                hh = hh + tmp
            mm = _swish(jnp.dot(hh, Wsk_r[...], preferred_element_type=F32)
                        + bsk_r[...]) + m_r[:, sl]
            for rr in range(NAFT):
                tmp = _swish(jnp.dot(mm, Wr2_r[2 * rr],
                                     preferred_element_type=F32)
                             + br2_r[2 * rr:2 * rr + 1, 0, :])
                tmp = _swish(jnp.dot(tmp, Wr2_r[2 * rr + 1],
                                     preferred_element_type=F32)
                             + br2_r[2 * rr + 1:2 * rr + 2, 0, :])
                mm = mm + tmp
            mouts.append(mm)
            rbf = geo2_r[:, 16 * h + 3:16 * h + 9]
            gouts.append(
                jnp.dot(rbf, Wro_r[...], preferred_element_type=F32) * mm)
        mout_r[...] = jnp.concatenate(mouts, axis=1)
        g_r[...] = jnp.concatenate(gouts, axis=1)

    bspec = lambda D: pl.BlockSpec((BH, D), lambda i: (i, 0))
    wspec = lambda s: pl.BlockSpec(s, lambda i: tuple(0 for _ in s))
    return pl.pallas_call(
        body,
        grid=grid,
        in_specs=[bspec(128), bspec(128), bspec(128), bspec(32),
                  wspec((2 * NBEF, EMB, EMB)), wspec((2 * NBEF, 1, EMB)),
                  wspec((EMB, EMB)), wspec((1, EMB)),
                  wspec((2 * NAFT, EMB, EMB)), wspec((2 * NAFT, 1, EMB)),
                  wspec((NRAD, EMB))],
        out_specs=[bspec(128), bspec(128)],
        out_shape=[jax.ShapeDtypeStruct((E2, 128), F32),
                   jax.ShapeDtypeStruct((E2, 128), F32)],
    )(xji, agg, m, geo2, Wr1, br1, Wsk, bsk, Wr2, br2, Wro)


def _tc_out(t0, t1, t2, Wd, bd, Wo, BN):
    N = t0.shape[0]
    grid = (N // BN,)

    def body(t0_r, t1_r, t2_r, Wd_r, bd_r, Wo_r, P_r):
        acc = jnp.zeros((t0_r.shape[0], NT), F32)
        for i, t_r in enumerate((t0_r, t1_r, t2_r)):
            t = t_r[...]
            for j in range(NDENSE):
                t = _swish(jnp.dot(t, Wd_r[i, j],
                                   preferred_element_type=F32)
                           + bd_r[i, j:j + 1, :].reshape(1, EMB))
            acc = acc + jnp.dot(t, Wo_r[i], preferred_element_type=F32)
        P_r[...] = acc

    bspec = lambda D: pl.BlockSpec((BN, D), lambda i: (i, 0))
    wspec = lambda s: pl.BlockSpec(s, lambda i: tuple(0 for _ in s))
    return pl.pallas_call(
        body,
        grid=grid,
        in_specs=[bspec(EMB), bspec(EMB), bspec(EMB),
                  wspec((NB + 1, NDENSE, EMB, EMB)),
                  wspec((NB + 1, NDENSE, EMB)),
                  wspec((NB + 1, EMB, NT))],
        out_specs=bspec(NT),
        out_shape=jax.ShapeDtypeStruct((N, NT), F32),
    )(t0, t1, t2, Wd, bd, Wo)


def kernel(pos, freq, emb_table, W_emb, b_emb, W_rbf_int, W_sbf_int, W_ji,
           b_ji, W_kj, b_kj, W_bilin, W_res1, b_res1, W_skip, b_skip, W_res2,
           b_res2, W_rbf_out, W_dense_out, b_dense_out, W_out, Z, edge_index,
           triplet_kj, triplet_ji):
    N = pos.shape[0]
    E = edge_index.shape[1]
    T = triplet_kj.shape[0]
    BN = 2000

    src = edge_index[0].astype(jnp.int32)
    dst = edge_index[1].astype(jnp.int32)
    kj = triplet_kj.astype(jnp.int32)
    ji = triplet_ji.astype(jnp.int32)
    kj2 = jnp.where(kj < E // 2, 2 * kj, 2 * (kj - E // 2) + 1)
    ji2 = jnp.where(ji < E // 2, 2 * ji, 2 * (ji - E // 2) + 1)

    NPAD = 10240
    Zp = jnp.pad(Z.astype(jnp.int32), (0, NPAD - N))
    posP = jnp.pad(pos.astype(F32), ((0, NPAD - N), (0, 8 - 3)))
    emb_p = jnp.pad(emb_table.astype(F32), ((0, 96 - 95), (0, 0)))
    W_src = W_emb[:EMB]
    W_dst = W_emb[EMB:2 * EMB]
    W3 = W_emb[2 * EMB:]
    b_emb2 = b_emb.reshape(1, EMB)
    Wcat = jnp.concatenate([W_sbf_int[0], W_sbf_int[1]], axis=1)
    Wb_t = jnp.transpose(W_bilin, (0, 2, 1, 3))
    b_ji2 = b_ji.reshape(NB, 1, EMB)
    b_kj2 = b_kj.reshape(NB, 1, EMB)
    b_res1_2 = b_res1.reshape(NB, 2 * NBEF, 1, EMB)
    b_skip2 = b_skip.reshape(NB, 1, EMB)
    b_res2_2 = b_res2.reshape(NB, 2 * NAFT, 1, EMB)

    A_tab, B_tab = _tc_emb_tables(emb_p, W_src, W_dst)

    nodeA, nodeB = _sc_gather((A_tab, B_tab), (Zp, Zp), (320, 320))

    gA, gB, ps, pd = _sc_gather(
        (nodeA, nodeB, posP, posP), (src, dst, src, dst),
        (200, 200, 1000, 1000), folds=(True, True, True, True))
    gA = gA.reshape(E // 2, 128)
    gB = gB.reshape(E // 2, 128)
    psd = jnp.concatenate(
        [ps.reshape(E // 2, 16), pd.reshape(E // 2, 16)], axis=1)

    m, geo2, g0 = _tc_edge(gA, gB, psd,
                           freq.reshape(1, NRAD).astype(F32),
                           W3, b_emb2, W_rbf_out[0])
    geom = geo2.reshape(E, 16)

    gkj, gji = _sc_gather((geom, geom), (kj2, ji2), (1000, 1000),
                          folds=(True, True))

    sbfw = _tc_sbfw(gkj, gji, Wcat)

    gs = [g0]
    for i in range(NB):
        xji, xkj = _tc_pre(m, geo2, W_ji[i], b_ji2[i], W_kj[i],
                           b_kj2[i], W_rbf_int[i])
        (xk,) = _sc_gather((xkj.reshape(E, 64),), (kj2,), (200,),
                           folds=(True,))
        xk = xk.reshape(T // 2, 128)
        xt = _tc_bilinear(xk, sbfw, Wb_t[i], NBIL * i)
        agg = _sc_segment_sum_folded(xt, ji, E, C=2000)
        m, g = _tc_post(xji, agg, m, geo2, W_res1[i], b_res1_2[i],
                        W_skip[i], b_skip2[i], W_res2[i], b_res2_2[i],
                        W_rbf_out[i + 1])
        gs.append(g)

    t0, t1, t2 = _sc_segment_sum3(gs, dst, NPAD, C=200)
    t0, t1, t2 = t0[:N], t1[:N], t2[:N]

    return _tc_out(t0, t1, t2, W_dense_out, b_dense_out, W_out, BN)
